# Initial kernel scaffold; baseline (speedup 1.0000x reference)
import jax
import jax.numpy as jnp
from jax.experimental import pallas as pl


def kernel(fragment_coordinates, fragment_cellxgene_ix, fragment_gene_ix, gene_ix, W_frag, W_expr, bias_expr):
    raise NotImplementedError("write your pallas kernel here")


# speedup vs baseline: 2.2138x; 2.2138x over previous
import functools

import numpy as np
import jax
import jax.numpy as jnp
from jax import lax
from jax.experimental import pallas as pl
from jax.experimental.pallas import tpu as pltpu
from jax.experimental.pallas import tpu_sc as plsc

N_FRAG = 800000
N_GENES = 100
CELL_N = 8000
GENE_N = 100
NF = 10
ED = 5
NSEG = CELL_N * GENE_N


R = 1600
GRID = N_FRAG // R


def _tc_body(c0_ref, c1_ref, g_ref, cxg_ref, f_ref, sh_ref, w2t_ref,
             weff_ref, sel_ref, out_ref):
    c0 = c0_ref[:, :]
    c1 = c1_ref[:, :]
    cc = jnp.concatenate(
        [jnp.broadcast_to(c0, (R, 2 * NF)), jnp.broadcast_to(c1, (R, 2 * NF))],
        axis=1)
    emb = jnp.sin(cc * f_ref[:, :] + sh_ref[:, :])
    g = g_ref[:, :]
    oh = (lax.broadcasted_iota(jnp.int32, (R, N_GENES), 1) == g
          ).astype(jnp.float32)
    wg = jnp.dot(oh, w2t_ref[:, :],
                 preferred_element_type=jnp.float32)
    emb5 = jnp.concatenate([emb] * ED, axis=1)
    p = emb5 * wg
    s = jnp.dot(p, sel_ref[:, :], preferred_element_type=jnp.float32)
    fe = jnp.maximum(s, 0.0)
    gp = cxg_ref[:, :] % GENE_N
    ohp = (lax.broadcasted_iota(jnp.int32, (R, GENE_N), 1) == gp
           ).astype(jnp.float32)
    wrow = jnp.dot(ohp, weff_ref[:, :],
                   preferred_element_type=jnp.float32)
    out_ref[:, :] = jnp.sum(fe * wrow, axis=1, keepdims=True)


def _stage1(c0, c1, g2, cxg2, freqs2, shifts2, w2t, weff, sel):
    full = lambda shape: pl.BlockSpec(shape, lambda i: (0, 0))
    blk = lambda shape: pl.BlockSpec(shape, lambda i: (i, 0))
    return pl.pallas_call(
        _tc_body,
        grid=(GRID,),
        in_specs=[
            blk((R, 1)), blk((R, 1)), blk((R, 1)), blk((R, 1)),
            full((1, 4 * NF)), full((1, 4 * NF)),
            full((N_GENES, 4 * NF * ED)), full((N_GENES, ED)),
            full((4 * NF * ED, ED)),
        ],
        out_specs=blk((R, 1)),
        out_shape=jax.ShapeDtypeStruct((N_FRAG, 1), jnp.float32),
    )(c0, c1, g2, cxg2, freqs2, shifts2, w2t, weff, sel)


NC = 2
NS = 16
CH = 10000
FRAG_PER_TILE = N_FRAG // NS
N_CHUNK = FRAG_PER_TILE // CH
SEG_PER_CORE = NSEG // NC
SEG_PER_TILE = SEG_PER_CORE // NS

@functools.cache
def _get_sc_seg():
    mesh = plsc.VectorSubcoreMesh(
        core_axis_name="c", subcore_axis_name="s",
        num_cores=NC, num_subcores=NS)
    return functools.partial(
        pl.kernel,
        mesh=mesh,
        out_type=jax.ShapeDtypeStruct((NSEG,), jnp.float32),
        scratch_types=[
            pltpu.VMEM((CH,), jnp.int32),
            pltpu.VMEM((CH,), jnp.float32),
            pltpu.VMEM((CH,), jnp.float32),
            pltpu.VMEM((CH,), jnp.float32),
            pltpu.VMEM((800,), jnp.float32),
            pltpu.VMEM_SHARED((SEG_PER_CORE,), jnp.float32),
            pltpu.VMEM_SHARED((SEG_PER_CORE,), jnp.float32),
        ],
    )(_sc_seg_body)


def _sc_seg_body(v_hbm, ids_hbm, bias_hbm, out_hbm, idx_b, val_b, ones_b,
                 r_out, bias_v, sums_sh, cnts_sh):
    cid = lax.axis_index("c")
    sid = lax.axis_index("s")
    core_base = cid * SEG_PER_CORE
    sbase = sid * SEG_PER_TILE

    def _zf(j, _):
        val_b[pl.ds(j * 16, 16)] = jnp.zeros((16,), jnp.float32)
        return 0
    lax.fori_loop(0, CH // 16, _zf, 0)
    for off in range(0, SEG_PER_TILE, CH):
        sz = min(CH, SEG_PER_TILE - off)
        pltpu.sync_copy(val_b.at[pl.ds(0, sz)],
                        sums_sh.at[pl.ds(sbase + off, sz)])
        pltpu.sync_copy(val_b.at[pl.ds(0, sz)],
                        cnts_sh.at[pl.ds(sbase + off, sz)])
    pltpu.sync_copy(bias_hbm, bias_v)
    plsc.subcore_barrier()

    frag_base = sid * FRAG_PER_TILE
    for ch in range(N_CHUNK):
        off = frag_base + ch * CH
        pltpu.sync_copy(ids_hbm.at[pl.ds(off, CH)], idx_b)
        pltpu.sync_copy(v_hbm.at[pl.ds(off, CH)], val_b)

        def _prep(j, _):
            sl = pl.ds(j * 16, 16)
            loc = idx_b[sl] - core_base
            inr = (loc >= 0) & (loc < SEG_PER_CORE)
            idx_b[sl] = jnp.where(inr, loc, 0)
            val_b[sl] = jnp.where(inr, val_b[sl], 0.0)
            ones_b[sl] = jnp.where(inr, 1.0, 0.0)
            return 0
        lax.fori_loop(0, CH // 16, _prep, 0)
        pltpu.sync_copy(val_b, sums_sh.at[idx_b], add=True)
        pltpu.sync_copy(ones_b, cnts_sh.at[idx_b], add=True)
    plsc.subcore_barrier()

    seg0 = core_base + sbase
    for off in range(0, SEG_PER_TILE, CH):
        sz = min(CH, SEG_PER_TILE - off)
        pltpu.sync_copy(sums_sh.at[pl.ds(sbase + off, sz)],
                        val_b.at[pl.ds(0, sz)])
        pltpu.sync_copy(cnts_sh.at[pl.ds(sbase + off, sz)],
                        ones_b.at[pl.ds(0, sz)])

        def _rd(j, _, sz=sz, off=off):
            base = jnp.minimum(j * 16, sz - 16)
            sl = pl.ds(base, 16)
            s = val_b[sl]
            c = ones_b[sl]
            b = bias_v[pl.ds((seg0 + off + base) % 400, 16)]
            r_out[sl] = s / jnp.maximum(c, 1.0) + b
            return 0
        lax.fori_loop(0, (sz + 15) // 16, _rd, 0)
        pltpu.sync_copy(r_out.at[pl.ds(0, sz)],
                        out_hbm.at[pl.ds(seg0 + off, sz)])


_FREQS = np.repeat(
    np.array([1.0 / 1000.0 ** (2.0 * i / NF) for i in range(1, NF + 1)],
             dtype=np.float32), 2)
_SHIFTS = np.tile(np.array([0.0, np.pi / 2.0], dtype=np.float32), NF)
_FREQS2 = np.concatenate([_FREQS, _FREQS])[None, :]
_SHIFTS2 = np.concatenate([_SHIFTS, _SHIFTS])[None, :]
_SEL = np.kron(np.eye(ED, dtype=np.float32),
               np.ones((4 * NF, 1), dtype=np.float32))


def kernel(fragment_coordinates, fragment_cellxgene_ix, fragment_gene_ix,
           gene_ix, W_frag, W_expr, bias_expr):
    c0 = fragment_coordinates[:, 0:1]
    c1 = fragment_coordinates[:, 1:2]
    g2 = fragment_gene_ix.reshape(N_FRAG, 1)
    cxg2 = fragment_cellxgene_ix.reshape(N_FRAG, 1)
    w2t = W_frag.transpose(0, 2, 1).reshape(N_GENES, 4 * NF * ED)
    weff = W_expr[gene_ix]
    bias_eff = bias_expr[gene_ix]
    bias_pad = jnp.tile(bias_eff, 8)

    v2d = _stage1(c0, c1, g2, cxg2, jnp.asarray(_FREQS2),
                  jnp.asarray(_SHIFTS2), w2t, weff, jnp.asarray(_SEL))
    out_flat = _get_sc_seg()(v2d.reshape(N_FRAG), fragment_cellxgene_ix,
                             bias_pad)
    return out_flat.reshape(CELL_N, GENE_N)
